# baseline (speedup 1.0000x reference)
import jax
import jax.numpy as jnp
from jax.experimental import pallas as pl

_N = 512
_D = 64
_DEPTH = 6
_F32 = jnp.float32
_HI = jax.lax.Precision.HIGHEST


def _exact_rowsum(x):
    s = x
    c = jnp.zeros_like(x)
    while s.shape[1] > 1:
        h = s.shape[1] // 2
        a, b = s[:, :h], s[:, h:]
        t = a + b
        bp = t - a
        err = (a - (t - bp)) + (b - bp)
        c = c[:, :h] + c[:, h:] + err
        s = t
    return s + c


def _body(x_ref, et_ref, at_ref, w0_ref, b0_ref, wq_ref, bq_ref, wk_ref,
          bk_ref, wv_ref, bv_ref, ws_ref, bs_ref, we_ref, be_ref, o_ref):
    x0 = x_ref[0]
    ewt = et_ref[0] * at_ref[0]
    x = None
    for j in range(_DEPTH):
        we_row = we_ref[j]
        be_row = be_ref[j]
        if j == 0:
            q = x0 * w0_ref[0] + b0_ref[0]
            k = x0 * w0_ref[1] + b0_ref[1]
            v = x0 * w0_ref[2] + b0_ref[2]
            skip = x0 * w0_ref[3] + b0_ref[3]
        else:
            i = j - 1
            q = jnp.dot(x, wq_ref[i], preferred_element_type=_F32) + bq_ref[i]
            k = jnp.dot(x, wk_ref[i], preferred_element_type=_F32) + bk_ref[i]
            v = jnp.dot(x, wv_ref[i], preferred_element_type=_F32) + bv_ref[i]
            skip = jnp.dot(x, ws_ref[i], preferred_element_type=_F32) + bs_ref[i]
        c = jnp.dot(q, we_row[:, None], preferred_element_type=_F32,
                    precision=_HI)
        d = jnp.dot(q, be_row[:, None], preferred_element_type=_F32,
                    precision=_HI)
        s_mat = jax.lax.dot_general(q, k, (((1,), (1,)), ((), ())),
                                    preferred_element_type=_F32,
                                    precision=_HI)
        s_mat = (s_mat + ewt * c + d) * 0.125
        m = jnp.max(s_mat, axis=1, keepdims=True)
        p = jnp.exp(s_mat - m)
        ssum = _exact_rowsum(p)
        r = 1.0 / (ssum + 1e-16)
        out1 = jnp.dot(p, v, preferred_element_type=_F32, precision=_HI) * r
        wsum = _exact_rowsum(p * ewt) * r
        sal = ssum * r
        x = out1 + wsum * we_row[None, :] + sal * be_row[None, :] + skip
    o_ref[0] = x


def kernel(node, edge, A, params):
    b, n, _ = node.shape

    w0 = jnp.stack([params[0][nm][0] for nm in ("Wq", "Wk", "Wv", "Ws")])
    b0 = jnp.stack([params[0][nm] for nm in ("bq", "bk", "bv", "bs")])
    wq = jnp.stack([p["Wq"] for p in params[1:]])
    wk = jnp.stack([p["Wk"] for p in params[1:]])
    wv = jnp.stack([p["Wv"] for p in params[1:]])
    ws = jnp.stack([p["Ws"] for p in params[1:]])
    bq = jnp.stack([p["bq"] for p in params[1:]])
    bk = jnp.stack([p["bk"] for p in params[1:]])
    bv = jnp.stack([p["bv"] for p in params[1:]])
    bs = jnp.stack([p["bs"] for p in params[1:]])
    we = jnp.stack([p["We"][0] for p in params])
    be = jnp.stack([p["be"] for p in params])

    edge_t = jnp.swapaxes(edge, 1, 2)
    a_t = jnp.swapaxes(A, 1, 2)

    full = lambda arr: pl.BlockSpec(arr.shape, lambda i: (0,) * arr.ndim)
    out = pl.pallas_call(
        _body,
        grid=(b,),
        in_specs=[
            pl.BlockSpec((1, n, 1), lambda i: (i, 0, 0)),
            pl.BlockSpec((1, n, n), lambda i: (i, 0, 0)),
            pl.BlockSpec((1, n, n), lambda i: (i, 0, 0)),
            full(w0), full(b0),
            full(wq), full(bq), full(wk), full(bk),
            full(wv), full(bv), full(ws), full(bs),
            full(we), full(be),
        ],
        out_specs=pl.BlockSpec((1, n, _D), lambda i: (i, 0, 0)),
        out_shape=jax.ShapeDtypeStruct((b, n, _D), _F32),
    )(node, edge_t, a_t, w0, b0, wq, bq, wk, bk, wv, bv, ws, bs, we, be)
    return out


# speedup vs baseline: 1.8946x; 1.8946x over previous
import jax
import jax.numpy as jnp
from jax.experimental import pallas as pl

_N = 512
_D = 64
_DEPTH = 6
_F32 = jnp.float32
_HI = jax.lax.Precision.HIGHEST


def _body(x_ref, et_ref, at_ref, w0_ref, b0_ref, wq_ref, bq_ref, wk_ref,
          bk_ref, wv_ref, bv_ref, ws_ref, bs_ref, we_ref, be_ref, o_ref):
    x0 = x_ref[0]
    ewt = et_ref[0] * at_ref[0]
    x = None
    for j in range(_DEPTH):
        we_row = we_ref[j]
        be_row = be_ref[j]
        if j == 0:
            q = x0 * w0_ref[0] + b0_ref[0]
            k = x0 * w0_ref[1] + b0_ref[1]
            v = x0 * w0_ref[2] + b0_ref[2]
            skip = x0 * w0_ref[3] + b0_ref[3]
        else:
            i = j - 1
            q = jnp.dot(x, wq_ref[i], preferred_element_type=_F32) + bq_ref[i]
            k = jnp.dot(x, wk_ref[i], preferred_element_type=_F32) + bk_ref[i]
            v = jnp.dot(x, wv_ref[i], preferred_element_type=_F32) + bv_ref[i]
            skip = jnp.dot(x, ws_ref[i], preferred_element_type=_F32) + bs_ref[i]
        c = jnp.dot(q, we_row[:, None], preferred_element_type=_F32,
                    precision=_HI)
        d = jnp.dot(q, be_row[:, None], preferred_element_type=_F32,
                    precision=_HI)
        s_mat = jax.lax.dot_general(q, k, (((1,), (1,)), ((), ())),
                                    preferred_element_type=_F32,
                                    precision=_HI)
        s_mat = (s_mat + ewt * c + d) * 0.125
        m = jnp.max(s_mat, axis=1, keepdims=True)
        p = jnp.exp(s_mat - m)
        ssum = jnp.sum(p, axis=1, keepdims=True)
        r = 1.0 / (ssum + 1e-16)
        out1 = jnp.dot(p, v, preferred_element_type=_F32, precision=_HI) * r
        wsum = jnp.sum(p * ewt, axis=1, keepdims=True) * r
        sal = ssum * r
        x = out1 + wsum * we_row[None, :] + sal * be_row[None, :] + skip
    o_ref[0] = x


def kernel(node, edge, A, params):
    b, n, _ = node.shape

    w0 = jnp.stack([params[0][nm][0] for nm in ("Wq", "Wk", "Wv", "Ws")])
    b0 = jnp.stack([params[0][nm] for nm in ("bq", "bk", "bv", "bs")])
    wq = jnp.stack([p["Wq"] for p in params[1:]])
    wk = jnp.stack([p["Wk"] for p in params[1:]])
    wv = jnp.stack([p["Wv"] for p in params[1:]])
    ws = jnp.stack([p["Ws"] for p in params[1:]])
    bq = jnp.stack([p["bq"] for p in params[1:]])
    bk = jnp.stack([p["bk"] for p in params[1:]])
    bv = jnp.stack([p["bv"] for p in params[1:]])
    bs = jnp.stack([p["bs"] for p in params[1:]])
    we = jnp.stack([p["We"][0] for p in params])
    be = jnp.stack([p["be"] for p in params])

    edge_t = jnp.swapaxes(edge, 1, 2)
    a_t = jnp.swapaxes(A, 1, 2)

    full = lambda arr: pl.BlockSpec(arr.shape, lambda i: (0,) * arr.ndim)
    out = pl.pallas_call(
        _body,
        grid=(b,),
        in_specs=[
            pl.BlockSpec((1, n, 1), lambda i: (i, 0, 0)),
            pl.BlockSpec((1, n, n), lambda i: (i, 0, 0)),
            pl.BlockSpec((1, n, n), lambda i: (i, 0, 0)),
            full(w0), full(b0),
            full(wq), full(bq), full(wk), full(bk),
            full(wv), full(bv), full(ws), full(bs),
            full(we), full(be),
        ],
        out_specs=pl.BlockSpec((1, n, _D), lambda i: (i, 0, 0)),
        out_shape=jax.ShapeDtypeStruct((b, n, _D), _F32),
    )(node, edge_t, a_t, w0, b0, wq, bq, wk, bk, wv, bv, ws, bs, we, be)
    return out
